# baseline (speedup 1.0000x reference)
import functools

import jax
import jax.numpy as jnp
from jax import lax
from jax.experimental import pallas as pl
from jax.experimental.pallas import tpu as pltpu
from jax.experimental.pallas import tpu_sc as plsc

NUM_EMBEDDINGS = 1000000
EMBEDDING_DIM = 64
BATCH_SIZE = 16384

NUM_CORES = 2
NUM_SUBCORES = 16
NUM_WORKERS = NUM_CORES * NUM_SUBCORES
B_PER_W = BATCH_SIZE // NUM_WORKERS
CHUNK = 128
NCH = B_PER_W // CHUNK
LANES = 16


@jax.jit
def _sc_gather(idx, table2):
    mesh = plsc.VectorSubcoreMesh(core_axis_name="c", subcore_axis_name="s")

    @functools.partial(
        pl.kernel,
        out_type=jax.ShapeDtypeStruct((EMBEDDING_DIM, BATCH_SIZE),
                                      jnp.float32),
        mesh=mesh,
        scratch_types=[
            pltpu.VMEM((B_PER_W,), jnp.int32),
            pltpu.VMEM((B_PER_W,), jnp.int32),
            pltpu.VMEM((2, CHUNK, 2 * EMBEDDING_DIM), jnp.float32),
            pltpu.VMEM((EMBEDDING_DIM, B_PER_W), jnp.float32),
            pltpu.SemaphoreType.DMA,
        ],
        compiler_params=pltpu.CompilerParams(needs_layout_passes=False),
    )
    def k(idx_hbm, table_hbm, out_hbm, idx_v, tid_v, gath_v, rowsT_v, sem):
        wid = lax.axis_index("s") * NUM_CORES + lax.axis_index("c")
        base = wid * B_PER_W
        pltpu.sync_copy(idx_hbm.at[pl.ds(base, B_PER_W)], idx_v)

        for i in range(B_PER_W // LANES):
            tid_v[pl.ds(i * LANES, LANES)] = idx_v[pl.ds(i * LANES, LANES)] >> 1

        iota = lax.iota(jnp.int32, LANES)

        def fire(ch, buf):
            return pltpu.async_copy(
                table_hbm.at[tid_v.at[pl.ds(ch * CHUNK, CHUNK)]],
                gath_v.at[buf],
                sem,
            )

        def extract(ch, buf):
            for g in range(CHUNK // LANES):
                b_vec = g * LANES + iota
                raw = idx_v[pl.ds(ch * CHUNK + g * LANES, LANES)]
                d1_base = (raw & 1) * EMBEDDING_DIM
                for c in range(EMBEDDING_DIM):
                    val = plsc.load_gather(
                        gath_v.at[buf], [b_vec, d1_base + c]
                    )
                    rowsT_v[c, pl.ds(ch * CHUNK + g * LANES, LANES)] = val

        cp = fire(0, 0)
        for ch in range(NCH):
            nxt = None
            if ch + 1 < NCH:
                nxt = fire(ch + 1, (ch + 1) % 2)
            cp.wait()
            extract(ch, ch % 2)
            cp = nxt

        pltpu.sync_copy(rowsT_v, out_hbm.at[:, pl.ds(base, B_PER_W)])

    return k(idx, table2)


def kernel(inputs, embeddings):
    table2 = embeddings.reshape(NUM_EMBEDDINGS // 2, 2 * EMBEDDING_DIM)
    out_t = _sc_gather(inputs.astype(jnp.int32), table2)
    return out_t.T


# speedup vs baseline: 3.4278x; 3.4278x over previous
import functools

import jax
import jax.numpy as jnp
from jax import lax
from jax.experimental import pallas as pl
from jax.experimental.pallas import tpu as pltpu
from jax.experimental.pallas import tpu_sc as plsc

NUM_EMBEDDINGS = 1000000
EMBEDDING_DIM = 64
BATCH_SIZE = 16384

NUM_CORES = 2
NUM_SUBCORES = 16
NUM_WORKERS = NUM_CORES * NUM_SUBCORES
B_PER_W = BATCH_SIZE // NUM_WORKERS
G = 8
RING = 8
FLUSH_GROUPS = 4
NSG = B_PER_W // (G * FLUSH_GROUPS)
LANES = 16


@jax.jit
def _sc_gather(r_s, b3, table_t):
    mesh = plsc.VectorSubcoreMesh(core_axis_name="c", subcore_axis_name="s")

    @functools.partial(
        pl.kernel,
        out_type=jax.ShapeDtypeStruct((BATCH_SIZE, 2 * EMBEDDING_DIM),
                                      jnp.float32),
        mesh=mesh,
        scratch_types=[
            pltpu.VMEM((544,), jnp.int32),
            pltpu.VMEM((16, 32), jnp.int32),
            pltpu.VMEM((RING, EMBEDDING_DIM, 128), jnp.float32),
            pltpu.VMEM((FLUSH_GROUPS * G, 128), jnp.float32),
            pltpu.SemaphoreType.DMA,
            pltpu.SemaphoreType.DMA,
        ],
        compiler_params=pltpu.CompilerParams(
            needs_layout_passes=False, disable_bounds_checks=True
        ),
    )
    def k(r_hbm, b3_hbm, table_hbm, out_hbm, r_v, b_v, ring_v, rows_v,
          sem, sem2):
        wid = lax.axis_index("s") * NUM_CORES + lax.axis_index("c")
        base = wid * B_PER_W
        pltpu.sync_copy(r_hbm.at[pl.ds(base, 16)], r_v.at[pl.ds(0, 16)])
        pltpu.sync_copy(r_hbm.at[pl.ds(base, B_PER_W)],
                        r_v.at[pl.ds(16, B_PER_W)])
        pltpu.sync_copy(b3_hbm.at[pl.ds(16 * wid, 16)], b_v)

        iota = lax.iota(jnp.int32, LANES)
        active = iota < G
        one = jnp.where(active, 1, 0)

        def do_group(g, q):
            r16 = r_v[pl.ds(16 + G * g, LANES)]
            rprev = r_v[pl.ds(15 + G * g, LANES)]
            jv = r16 >> 7
            newm = ((jv != (rprev >> 7)) | (iota == 0)) & active
            rank = plsc.cumsum(jnp.where(newm, one, 0)) - 1
            n_new = jnp.sum(jnp.where(newm, one, 0))
            col = r16 & 127

            def fire(k_, _):
                jk = jnp.sum(jnp.where(newm & (rank == k_), jv, 0))
                off = pl.multiple_of(jk * 128, 128)
                pltpu.async_copy(
                    table_hbm.at[:, pl.ds(off, 128)], ring_v.at[k_], sem
                )
                return 0

            lax.fori_loop(0, n_new, fire, 0)

            def drain(k_, _):
                pltpu.make_async_copy(
                    table_hbm.at[:, pl.ds(0, 128)], ring_v.at[k_], sem
                ).wait()
                return 0

            lax.fori_loop(0, n_new, drain, 0)

            row16 = q * G + iota
            for c in range(EMBEDDING_DIM):
                cvec = jnp.full((LANES,), c, dtype=jnp.int32)
                val = plsc.load_gather(ring_v, [rank, cvec, col])
                plsc.store_scatter(rows_v, [row16, cvec], val, mask=active)

        def sg_body(sg, _):
            for q in range(FLUSH_GROUPS):
                do_group(sg * FLUSH_GROUPS + q, q)
            pltpu.async_copy(rows_v, out_hbm.at[b_v.at[sg]], sem2).wait()
            return 0

        lax.fori_loop(0, NSG, sg_body, 0)

    return k(r_s, b3, table_t)


def kernel(inputs, embeddings):
    idx = inputs.astype(jnp.int32)
    pos = jnp.arange(BATCH_SIZE, dtype=jnp.int32)
    r_s, b_s = lax.sort((idx, pos), num_keys=1)
    out = _sc_gather(r_s, b_s.reshape(512, 32), embeddings.T)
    return out[:, :EMBEDDING_DIM]


# speedup vs baseline: 3.9922x; 1.1646x over previous
import functools

import jax
import jax.numpy as jnp
from jax import lax
from jax.experimental import pallas as pl
from jax.experimental.pallas import tpu as pltpu
from jax.experimental.pallas import tpu_sc as plsc

NUM_EMBEDDINGS = 1000000
EMBEDDING_DIM = 64
BATCH_SIZE = 16384

NUM_PAIRS = 16
B_PER_P = BATCH_SIZE // NUM_PAIRS
HALF_D = EMBEDDING_DIM // 2
G = 8
NGROUPS = B_PER_P // G
RING = 8
FLUSH_GROUPS = 4
NSG = NGROUPS // FLUSH_GROUPS
LANES = 16
MAX_J = (NUM_EMBEDDINGS - 1) // 128


@jax.jit
def _sc_gather(r_s, b3, table_t):
    mesh = plsc.VectorSubcoreMesh(core_axis_name="c", subcore_axis_name="s")

    @functools.partial(
        pl.kernel,
        out_type=jax.ShapeDtypeStruct((2, BATCH_SIZE, 2 * EMBEDDING_DIM),
                                      jnp.float32),
        mesh=mesh,
        scratch_types=[
            pltpu.VMEM((1072,), jnp.int32),
            pltpu.VMEM((32, 32), jnp.int32),
            pltpu.VMEM((2 * RING, HALF_D, 128), jnp.float32),
            pltpu.VMEM((FLUSH_GROUPS * G, 128), jnp.float32),
            pltpu.SemaphoreType.DMA,
            pltpu.SemaphoreType.DMA,
        ],
        compiler_params=pltpu.CompilerParams(
            needs_layout_passes=False, disable_bounds_checks=True
        ),
    )
    def k(r_hbm, b3_hbm, table_hbm, out_hbm, r_v, b_v, ring_v, rows_v,
          sem, sem2):
        pair = lax.axis_index("s")
        h = lax.axis_index("c")
        base = pair * B_PER_P
        pltpu.sync_copy(r_hbm.at[pl.ds(base, 16)], r_v.at[pl.ds(0, 16)])
        pltpu.sync_copy(r_hbm.at[pl.ds(base, B_PER_P)],
                        r_v.at[pl.ds(16, B_PER_P)])
        pltpu.sync_copy(b3_hbm.at[pl.ds(32 * pair, 32)], b_v)

        iota = lax.iota(jnp.int32, LANES)
        active = iota < G
        one = jnp.where(active, 1, 0)
        d0 = h * HALF_D

        def group_info(g):
            r16 = r_v[pl.ds(16 + G * g, LANES)]
            rprev = r_v[pl.ds(15 + G * g, LANES)]
            jv = r16 >> 7
            newm = ((jv != (rprev >> 7)) | (iota == 0)) & active
            rank = plsc.cumsum(jnp.where(newm, one, 0)) - 1
            n_new = jnp.sum(jnp.where(newm, one, 0))
            return jv, newm, rank, n_new, r16 & 127

        def fire(g):
            jv, newm, rank, n_new, _ = group_info(g)
            sbase = (g % 2) * RING

            def body(k_, _):
                jk = jnp.minimum(
                    jnp.sum(jnp.where(newm & (rank == k_), jv, 0)), MAX_J
                )
                off = pl.multiple_of(jk * 128, 128)
                pltpu.async_copy(
                    table_hbm.at[pl.ds(d0, HALF_D), pl.ds(off, 128)],
                    ring_v.at[sbase + k_], sem,
                )
                return 0

            lax.fori_loop(0, n_new, body, 0)

        def drain_extract(g, q):
            _, _, rank, n_new, col = group_info(g)
            sbase = (g % 2) * RING

            def body(k_, _):
                pltpu.make_async_copy(
                    table_hbm.at[pl.ds(0, HALF_D), pl.ds(0, 128)],
                    ring_v.at[sbase + k_], sem,
                ).wait()
                return 0

            lax.fori_loop(0, n_new, body, 0)

            row16 = q * G + iota
            slot16 = sbase + rank
            for c in range(HALF_D):
                cvec = jnp.full((LANES,), c, dtype=jnp.int32)
                val = plsc.load_gather(ring_v, [slot16, cvec, col])
                plsc.store_scatter(rows_v, [row16, cvec], val, mask=active)

        fire(0)

        def sg_body(sg, _):
            for q in range(FLUSH_GROUPS):
                g = sg * FLUSH_GROUPS + q
                fire(g + 1)
                drain_extract(g, q)
            pltpu.async_copy(
                rows_v, out_hbm.at[h].at[b_v.at[sg]], sem2
            ).wait()
            return 0

        lax.fori_loop(0, NSG - 1, sg_body, 0)

        for q in range(FLUSH_GROUPS):
            g = (NSG - 1) * FLUSH_GROUPS + q
            if g + 1 < NGROUPS:
                fire(g + 1)
            drain_extract(g, q)
        pltpu.async_copy(
            rows_v, out_hbm.at[h].at[b_v.at[NSG - 1]], sem2
        ).wait()

    return k(r_s, b3, table_t)


def kernel(inputs, embeddings):
    idx = inputs.astype(jnp.int32)
    pos = jnp.arange(BATCH_SIZE, dtype=jnp.int32)
    r_s, b_s = lax.sort((idx, pos), num_keys=1)
    out = _sc_gather(r_s, b_s.reshape(512, 32), embeddings.T)
    return jnp.concatenate([out[0, :, :HALF_D], out[1, :, :HALF_D]], axis=1)


# speedup vs baseline: 3.9979x; 1.0014x over previous
import functools

import jax
import jax.numpy as jnp
from jax import lax
from jax.experimental import pallas as pl
from jax.experimental.pallas import tpu as pltpu
from jax.experimental.pallas import tpu_sc as plsc

NUM_EMBEDDINGS = 1000000
EMBEDDING_DIM = 64
BATCH_SIZE = 16384

NUM_PAIRS = 16
B_PER_P = BATCH_SIZE // NUM_PAIRS
HALF_D = EMBEDDING_DIM // 2
G = 8
NGROUPS = B_PER_P // G
RING = 8
FLUSH_GROUPS = 4
NSG = NGROUPS // FLUSH_GROUPS
LANES = 16
MAX_J = (NUM_EMBEDDINGS - 1) // 128


@jax.jit
def _sc_gather(r_s, b3, table_t):
    mesh = plsc.VectorSubcoreMesh(core_axis_name="c", subcore_axis_name="s")

    @functools.partial(
        pl.kernel,
        out_type=jax.ShapeDtypeStruct((2, BATCH_SIZE, 2 * EMBEDDING_DIM),
                                      jnp.float32),
        mesh=mesh,
        scratch_types=[
            pltpu.VMEM((1072,), jnp.int32),
            pltpu.VMEM((32, 32), jnp.int32),
            pltpu.VMEM((2 * RING, HALF_D, 128), jnp.float32),
            pltpu.VMEM((2, FLUSH_GROUPS * G, 128), jnp.float32),
            pltpu.SemaphoreType.DMA,
            pltpu.SemaphoreType.DMA,
        ],
        compiler_params=pltpu.CompilerParams(
            needs_layout_passes=False, disable_bounds_checks=True
        ),
    )
    def k(r_hbm, b3_hbm, table_hbm, out_hbm, r_v, b_v, ring_v, rows_v,
          sem, sem2):
        pair = lax.axis_index("s")
        h = lax.axis_index("c")
        base = pair * B_PER_P
        pltpu.sync_copy(r_hbm.at[pl.ds(base, 16)], r_v.at[pl.ds(0, 16)])
        pltpu.sync_copy(r_hbm.at[pl.ds(base, B_PER_P)],
                        r_v.at[pl.ds(16, B_PER_P)])
        pltpu.sync_copy(b3_hbm.at[pl.ds(32 * pair, 32)], b_v)

        iota = lax.iota(jnp.int32, LANES)
        active = iota < G
        one = jnp.where(active, 1, 0)
        d0 = h * HALF_D

        def group_info(g):
            r16 = r_v[pl.ds(16 + G * g, LANES)]
            rprev = r_v[pl.ds(15 + G * g, LANES)]
            jv = r16 >> 7
            newm = ((jv != (rprev >> 7)) | (iota == 0)) & active
            rank = plsc.cumsum(jnp.where(newm, one, 0)) - 1
            n_new = jnp.sum(jnp.where(newm, one, 0))
            return jv, newm, rank, n_new, r16 & 127

        def fire(g):
            jv, newm, rank, n_new, _ = group_info(g)
            sbase = (g % 2) * RING

            def body(k_, _):
                jk = jnp.minimum(
                    jnp.sum(jnp.where(newm & (rank == k_), jv, 0)), MAX_J
                )
                off = pl.multiple_of(jk * 128, 128)
                pltpu.async_copy(
                    table_hbm.at[pl.ds(d0, HALF_D), pl.ds(off, 128)],
                    ring_v.at[sbase + k_], sem,
                )
                return 0

            lax.fori_loop(0, n_new, body, 0)

        def drain_extract(g, q):
            _, _, rank, n_new, col = group_info(g)
            sbase = (g % 2) * RING

            def body(k_, _):
                pltpu.make_async_copy(
                    table_hbm.at[pl.ds(0, HALF_D), pl.ds(0, 128)],
                    ring_v.at[sbase + k_], sem,
                ).wait()
                return 0

            lax.fori_loop(0, n_new, body, 0)

            row16 = q * G + iota
            slot16 = sbase + rank
            buf16 = jnp.full((LANES,), 0, dtype=jnp.int32) + ((g // FLUSH_GROUPS) % 2)
            for c in range(HALF_D):
                cvec = jnp.full((LANES,), c, dtype=jnp.int32)
                val = plsc.load_gather(ring_v, [slot16, cvec, col])
                plsc.store_scatter(rows_v, [buf16, row16, cvec], val,
                                   mask=active)

        def fire_scatter(sg):
            pltpu.async_copy(
                rows_v.at[sg % 2], out_hbm.at[h].at[b_v.at[sg]], sem2
            )

        def drain_scatter():
            pltpu.make_async_copy(
                rows_v.at[0], out_hbm.at[h, pl.ds(0, FLUSH_GROUPS * G)], sem2
            ).wait()

        fire(0)

        for q in range(FLUSH_GROUPS):
            fire(q + 1)
            drain_extract(q, q)
        fire_scatter(0)

        def sg_body(sg, _):
            for q in range(FLUSH_GROUPS):
                g = sg * FLUSH_GROUPS + q
                fire(g + 1)
                drain_extract(g, q)
            fire_scatter(sg)
            drain_scatter()
            return 0

        lax.fori_loop(1, NSG - 1, sg_body, 0)

        for q in range(FLUSH_GROUPS):
            g = (NSG - 1) * FLUSH_GROUPS + q
            if g + 1 < NGROUPS:
                fire(g + 1)
            drain_extract(g, q)
        fire_scatter(NSG - 1)
        drain_scatter()
        drain_scatter()

    return k(r_s, b3, table_t)


def kernel(inputs, embeddings):
    idx = inputs.astype(jnp.int32)
    pos = jnp.arange(BATCH_SIZE, dtype=jnp.int32)
    r_s, b_s = lax.sort((idx, pos), num_keys=1)
    out = _sc_gather(r_s, b_s.reshape(512, 32), embeddings.T)
    return jnp.concatenate([out[0, :, :HALF_D], out[1, :, :HALF_D]], axis=1)


# speedup vs baseline: 4.5631x; 1.1414x over previous
import functools

import jax
import jax.numpy as jnp
from jax import lax
from jax.experimental import pallas as pl
from jax.experimental.pallas import tpu as pltpu
from jax.experimental.pallas import tpu_sc as plsc

NUM_EMBEDDINGS = 1000000
EMBEDDING_DIM = 64
BATCH_SIZE = 16384

NUM_PAIRS = 16
B_PER_P = BATCH_SIZE // NUM_PAIRS
HALF_D = EMBEDDING_DIM // 2
G = 8
NGROUPS = B_PER_P // G
RING = 8
FLUSH_GROUPS = 4
NSG = NGROUPS // FLUSH_GROUPS
LANES = 16
MAX_J = (NUM_EMBEDDINGS - 1) // 128


@jax.jit
def _sc_gather(r_s, b3, table_t):
    mesh = plsc.VectorSubcoreMesh(core_axis_name="c", subcore_axis_name="s")

    @functools.partial(
        pl.kernel,
        out_type=jax.ShapeDtypeStruct((2, BATCH_SIZE, 2 * EMBEDDING_DIM),
                                      jnp.float32),
        mesh=mesh,
        scratch_types=[
            pltpu.VMEM((1072,), jnp.int32),
            pltpu.VMEM((32, 32), jnp.int32),
            pltpu.VMEM((3 * RING, HALF_D, 128), jnp.float32),
            pltpu.VMEM((2, FLUSH_GROUPS * G, 128), jnp.float32),
            pltpu.SemaphoreType.DMA,
            pltpu.SemaphoreType.DMA,
        ],
        compiler_params=pltpu.CompilerParams(
            needs_layout_passes=False, disable_bounds_checks=True
        ),
    )
    def k(r_hbm, b3_hbm, table_hbm, out_hbm, r_v, b_v, ring_v, rows_v,
          sem, sem2):
        pair = lax.axis_index("s")
        h = lax.axis_index("c")
        base = pair * B_PER_P
        pltpu.sync_copy(r_hbm.at[pl.ds(base, 16)], r_v.at[pl.ds(0, 16)])
        pltpu.sync_copy(r_hbm.at[pl.ds(base, B_PER_P)],
                        r_v.at[pl.ds(16, B_PER_P)])
        pltpu.sync_copy(b3_hbm.at[pl.ds(32 * pair, 32)], b_v)

        iota = lax.iota(jnp.int32, LANES)
        active = iota < G
        one = jnp.where(active, 1, 0)
        d0 = h * HALF_D

        def group_info(g):
            r16 = r_v[pl.ds(16 + G * g, LANES)]
            rprev = r_v[pl.ds(15 + G * g, LANES)]
            jv = r16 >> 7
            newm = ((jv != (rprev >> 7)) | (iota == 0)) & active
            rank = plsc.cumsum(jnp.where(newm, one, 0)) - 1
            n_new = jnp.sum(jnp.where(newm, one, 0))
            return jv, newm, rank, n_new, r16 & 127

        def fire(g):
            jv, newm, rank, n_new, _ = group_info(g)
            sbase = (g % 3) * RING

            def body(k_, _):
                jk = jnp.minimum(
                    jnp.sum(jnp.where(newm & (rank == k_), jv, 0)), MAX_J
                )
                off = pl.multiple_of(jk * 128, 128)
                pltpu.async_copy(
                    table_hbm.at[pl.ds(d0, HALF_D), pl.ds(off, 128)],
                    ring_v.at[sbase + k_], sem,
                )
                return 0

            lax.fori_loop(0, n_new, body, 0)

        def drain_extract(g, q):
            _, _, rank, n_new, col = group_info(g)
            sbase = (g % 3) * RING

            def body(k_, _):
                pltpu.make_async_copy(
                    table_hbm.at[pl.ds(0, HALF_D), pl.ds(0, 128)],
                    ring_v.at[sbase + k_], sem,
                ).wait()
                return 0

            lax.fori_loop(0, n_new, body, 0)

            row16 = q * G + iota
            slot16 = sbase + rank
            buf16 = jnp.full((LANES,), 0, dtype=jnp.int32) + ((g // FLUSH_GROUPS) % 2)
            for c in range(HALF_D):
                cvec = jnp.full((LANES,), c, dtype=jnp.int32)
                val = plsc.load_gather(ring_v, [slot16, cvec, col])
                plsc.store_scatter(rows_v, [buf16, row16, cvec], val,
                                   mask=active)

        def fire_scatter(sg):
            pltpu.async_copy(
                rows_v.at[sg % 2], out_hbm.at[h].at[b_v.at[sg]], sem2
            )

        def drain_scatter():
            pltpu.make_async_copy(
                rows_v.at[0], out_hbm.at[h, pl.ds(0, FLUSH_GROUPS * G)], sem2
            ).wait()

        fire(0)
        fire(1)

        for q in range(FLUSH_GROUPS):
            fire(q + 2)
            drain_extract(q, q)
        fire_scatter(0)

        def sg_body(sg, _):
            for q in range(FLUSH_GROUPS):
                g = sg * FLUSH_GROUPS + q
                fire(g + 2)
                drain_extract(g, q)
            fire_scatter(sg)
            drain_scatter()
            return 0

        lax.fori_loop(1, NSG - 1, sg_body, 0)

        for q in range(FLUSH_GROUPS):
            g = (NSG - 1) * FLUSH_GROUPS + q
            if g + 2 < NGROUPS:
                fire(g + 2)
            drain_extract(g, q)
        fire_scatter(NSG - 1)
        drain_scatter()
        drain_scatter()

    return k(r_s, b3, table_t)


def kernel(inputs, embeddings):
    idx = inputs.astype(jnp.int32)
    pos = jnp.arange(BATCH_SIZE, dtype=jnp.int32)
    r_s, b_s = lax.sort((idx, pos), num_keys=1)
    out = _sc_gather(r_s, b_s.reshape(512, 32), embeddings.T)
    return jnp.concatenate([out[0, :, :HALF_D], out[1, :, :HALF_D]], axis=1)
